# Initial kernel scaffold; baseline (speedup 1.0000x reference)
import jax
import jax.numpy as jnp
from jax.experimental import pallas as pl


def kernel(x, c, ei, ew, W1, b1, W2, b2, Wfc, bfc):
    raise NotImplementedError("write your pallas kernel here")


# speedup vs baseline: 4.4900x; 4.4900x over previous
import functools

import jax
import jax.numpy as jnp
from jax import lax
from jax.experimental import pallas as pl
from jax.experimental.pallas import tpu as pltpu
from jax.experimental.pallas import tpu_sc as plsc

N = 50000
E = 1600000
NC, NS = 2, 16
NW = NC * NS
NP = 50176
NSLICE = NP // NS
CH = 128
EW_PER = 50048
EP = NW * EW_PER
NCHUNK = EW_PER // CH
ECHUNK2 = EP // NS // CH
WB = 784
BR = 1792

_mesh = functools.partial(
    plsc.VectorSubcoreMesh, core_axis_name="c", subcore_axis_name="s"
)


def _f32(shape):
    return jax.ShapeDtypeStruct(shape, jnp.float32)


def _newton_rsqrt(d):
    i = plsc.bitcast(d, jnp.int32)
    i = jnp.int32(0x5F3759DF) - lax.shift_right_logical(i, 1)
    y = plsc.bitcast(i, jnp.float32)
    for _ in range(3):
        y = y * (1.5 - 0.5 * d * y * y)
    return y


def _make_l1_kernel():
    scratch = [
        pltpu.VMEM((CH,), jnp.int32),
        pltpu.VMEM((CH,), jnp.int32),
        pltpu.VMEM((CH,), jnp.float32),
        pltpu.VMEM((CH,), jnp.float32),
        pltpu.VMEM((CH,), jnp.float32),
        pltpu.VMEM((NP,), jnp.float32),
        pltpu.VMEM((NP,), jnp.float32),
        pltpu.VMEM((WB,), jnp.float32),
        pltpu.VMEM((WB,), jnp.float32),
        pltpu.VMEM((WB,), jnp.float32),
        pltpu.VMEM((WB * 4,), jnp.float32),
        pltpu.VMEM_SHARED((NP,), jnp.float32),
        pltpu.VMEM_SHARED((NP,), jnp.float32),
        pltpu.VMEM_SHARED((NP,), jnp.float32),
    ]

    @functools.partial(
        pl.kernel,
        out_type=(_f32((NC * NP * 4,)), _f32((EP,)), _f32((NP,))),
        mesh=_mesh(),
        scratch_types=scratch,
        compiler_params=pltpu.CompilerParams(needs_layout_passes=False),
    )
    def l1_kernel(row_hbm, col_hbm, ew_hbm, xf0_hbm, xf1_hbm, xf2_hbm,
                  zd_hbm, u_hbm, wn_hbm, dinv_hbm,
                  colv, rowv, ewv, wnv, msgv, dinv_v, xcol,
                  st0, st1, st2, wb4, acc0, acc1, acc2):
        cc = lax.axis_index("c")
        sid = lax.axis_index("s")
        w = cc * NS + sid
        ebase = w * EW_PER
        xf_refs = (xf0_hbm, xf1_hbm, xf2_hbm)
        accs = (acc0, acc1, acc2)
        iota = lax.iota(jnp.int32, 16)

        pltpu.sync_copy(zd_hbm, st0)
        for j in range(NSLICE // WB):
            off = sid * NSLICE + j * WB
            for a in accs:
                pltpu.sync_copy(st0, a.at[pl.ds(off, WB)])
        plsc.subcore_barrier()

        def dchunk(i, carry):
            base = sid * (EP // NS) + i * CH
            pltpu.sync_copy(col_hbm.at[pl.ds(base, CH)], colv)
            pltpu.sync_copy(ew_hbm.at[pl.ds(base, CH)], ewv)
            pltpu.sync_copy(ewv, acc0.at[colv], add=True)
            return carry

        lax.fori_loop(0, ECHUNK2, dchunk, 0)
        plsc.subcore_barrier()

        for j in range(NSLICE // WB):
            off = sid * NSLICE + j * WB
            pltpu.sync_copy(acc0.at[pl.ds(off, WB)], st0)

            def dgroup(g, carry):
                d = st0[pl.ds(g * 16, 16)] + 1.0
                st0[pl.ds(g * 16, 16)] = _newton_rsqrt(d)
                return carry

            lax.fori_loop(0, WB // 16, dgroup, 0)
            pltpu.sync_copy(st0, dinv_hbm.at[pl.ds(off, WB)])
        pltpu.sync_copy(zd_hbm, st0)
        for j in range(NSLICE // WB):
            pltpu.sync_copy(st0, acc0.at[pl.ds(sid * NSLICE + j * WB, WB)])
        plsc.subcore_barrier()
        pltpu.sync_copy(dinv_hbm, dinv_v)

        for f in range(3):
            pltpu.sync_copy(xf_refs[f], xcol)

            def chunk(i, carry):
                base = ebase + i * CH
                pltpu.sync_copy(col_hbm.at[pl.ds(base, CH)], colv)
                pltpu.sync_copy(row_hbm.at[pl.ds(base, CH)], rowv)
                if f == 0:
                    pltpu.sync_copy(ew_hbm.at[pl.ds(base, CH)], ewv)
                else:
                    pltpu.sync_copy(wn_hbm.at[pl.ds(base, CH)], wnv)
                for g in range(CH // 16):
                    rg = rowv[pl.ds(g * 16, 16)]
                    if f == 0:
                        cg = colv[pl.ds(g * 16, 16)]
                        eg = ewv[pl.ds(g * 16, 16)]
                        dr = plsc.load_gather(dinv_v, [rg])
                        dc = plsc.load_gather(dinv_v, [cg])
                        wn = dr * eg * dc
                        wnv[pl.ds(g * 16, 16)] = wn
                    else:
                        wn = wnv[pl.ds(g * 16, 16)]
                    xr = plsc.load_gather(xcol, [rg])
                    msgv[pl.ds(g * 16, 16)] = xr * wn
                if f == 0:
                    pltpu.sync_copy(wnv, wn_hbm.at[pl.ds(base, CH)])
                pltpu.sync_copy(msgv, accs[f].at[colv], add=True)
                return carry

            lax.fori_loop(0, NCHUNK, chunk, 0)
        plsc.subcore_barrier()

        for j in range(NSLICE // WB):
            off = sid * NSLICE + j * WB
            pltpu.sync_copy(acc0.at[pl.ds(off, WB)], st0)
            pltpu.sync_copy(acc1.at[pl.ds(off, WB)], st1)
            pltpu.sync_copy(acc2.at[pl.ds(off, WB)], st2)

            def agroup(g, carry):
                rr = (g * 16 + iota) * 4
                plsc.store_scatter(wb4, [rr], st0[pl.ds(g * 16, 16)])
                plsc.store_scatter(wb4, [rr + 1], st1[pl.ds(g * 16, 16)])
                plsc.store_scatter(wb4, [rr + 2], st2[pl.ds(g * 16, 16)])
                plsc.store_scatter(wb4, [rr + 3],
                                   jnp.zeros((16,), jnp.float32))
                return carry

            lax.fori_loop(0, WB // 16, agroup, 0)
            pltpu.sync_copy(
                wb4, u_hbm.at[pl.ds((cc * NP + off) * 4, WB * 4)]
            )

    return l1_kernel


def _make_l2_kernel(D, n_slices):
    ND = NP * D
    scratch = [
        pltpu.VMEM((CH,), jnp.int32),
        pltpu.VMEM((CH,), jnp.int32),
        pltpu.VMEM((CH,), jnp.float32),
        pltpu.VMEM((D, CH), jnp.int32),
        pltpu.VMEM((D, CH), jnp.int32),
        pltpu.VMEM((D * CH,), jnp.float32),
        pltpu.VMEM((D * CH,), jnp.float32),
        pltpu.VMEM((WB * D,), jnp.float32),
        pltpu.VMEM_SHARED((ND,), jnp.float32),
        pltpu.VMEM_SHARED((ND,), jnp.float32),
        pltpu.SemaphoreType.DMA,
    ]

    @functools.partial(
        pl.kernel,
        out_type=_f32((n_slices * NC * NP * D,)),
        mesh=_mesh(),
        scratch_types=scratch,
        compiler_params=pltpu.CompilerParams(needs_layout_passes=False),
    )
    def l2_kernel(row_hbm, col_hbm, wn_hbm, src_hbm, zeros_hbm, u_hbm,
                  colv, rowv, wnv, gidx, sidx, xg, msg, wbv, acc, src_sh,
                  sem):
        cc = lax.axis_index("c")
        sid = lax.axis_index("s")
        w = cc * NS + sid
        ebase = w * EW_PER

        for k in range(n_slices):
            pltpu.sync_copy(zeros_hbm, wbv)
            for j in range(NSLICE // WB):
                pltpu.sync_copy(
                    wbv, acc.at[pl.ds((sid * NSLICE + j * WB) * D, WB * D)]
                )
            for j in range(NSLICE // WB):
                off = (sid * NSLICE + j * WB) * D
                pltpu.sync_copy(src_hbm.at[pl.ds(k * NP * D + off, WB * D)],
                                wbv)
                pltpu.sync_copy(wbv, src_sh.at[pl.ds(off, WB * D)])
            plsc.subcore_barrier()

            def chunk(i, carry):
                base = ebase + i * CH
                pltpu.sync_copy(col_hbm.at[pl.ds(base, CH)], colv)
                pltpu.sync_copy(row_hbm.at[pl.ds(base, CH)], rowv)
                pltpu.sync_copy(wn_hbm.at[pl.ds(base, CH)], wnv)
                for f in range(D):
                    for g in range(CH // 16):
                        rv = rowv[pl.ds(g * 16, 16)]
                        cv = colv[pl.ds(g * 16, 16)]
                        gidx[f, pl.ds(g * 16, 16)] = rv * D + f
                        sidx[f, pl.ds(g * 16, 16)] = cv * D + f
                copies = [
                    pltpu.async_copy(
                        src_sh.at[gidx.at[f]],
                        xg.at[pl.ds(f * CH, CH)], sem,
                    )
                    for f in range(D)
                ]
                for cp in copies:
                    cp.wait()
                for f in range(D):
                    for g in range(CH // 16):
                        o = f * CH + g * 16
                        msg[pl.ds(o, 16)] = (
                            xg[pl.ds(o, 16)] * wnv[pl.ds(g * 16, 16)]
                        )
                for f in range(D):
                    pltpu.sync_copy(
                        msg.at[pl.ds(f * CH, CH)],
                        acc.at[sidx.at[f]], add=True,
                    )
                return carry

            lax.fori_loop(0, NCHUNK, chunk, 0)
            plsc.subcore_barrier()
            for j in range(NSLICE // WB):
                off = (sid * NSLICE + j * WB) * D
                pltpu.sync_copy(acc.at[pl.ds(off, WB * D)], wbv)
                pltpu.sync_copy(
                    wbv,
                    u_hbm.at[
                        pl.ds(k * NC * NP * D + cc * NP * D + off, WB * D)
                    ],
                )

    return l2_kernel


def _mm1_tc(u_part, src, dinv_col, W, b):
    def body(u_ref, s_ref, d_ref, w_ref, b_ref, o_ref):
        d2 = d_ref[...] * d_ref[...]
        u = u_ref[0] + u_ref[1] + d2 * s_ref[...]
        h = jnp.maximum(
            jnp.dot(u, w_ref[...], preferred_element_type=jnp.float32)
            + b_ref[...],
            0.0,
        )
        for k in range(8):
            o_ref[k] = h[:, 4 * k:4 * k + 4]

    return pl.pallas_call(
        body,
        grid=(NP // BR,),
        in_specs=[
            pl.BlockSpec((NC, BR, 4), lambda i: (0, i, 0)),
            pl.BlockSpec((BR, 4), lambda i: (i, 0)),
            pl.BlockSpec((BR, 1), lambda i: (i, 0)),
            pl.BlockSpec((4, 32), lambda i: (0, 0)),
            pl.BlockSpec((1, 32), lambda i: (0, 0)),
        ],
        out_specs=pl.BlockSpec((8, BR, 4), lambda i: (0, i, 0)),
        out_shape=_f32((8, NP, 4)),
    )(u_part, src, dinv_col, W, b)


def _mm2_tc(u_part, h1_sl, dinv_col, W2, b2, Wfc, bfc):
    def body(u_ref, s_ref, d_ref, w_ref, b_ref, wf_ref, bf_ref, o_ref):
        d2 = d_ref[...] * d_ref[...]
        u = jnp.concatenate(
            [u_ref[k, 0] + u_ref[k, 1] for k in range(8)], axis=1
        )
        h1 = jnp.concatenate([s_ref[k] for k in range(8)], axis=1)
        h2 = jnp.maximum(
            jnp.dot(u + d2 * h1, w_ref[...],
                    preferred_element_type=jnp.float32) + b_ref[...],
            0.0,
        )
        o_ref[...] = (
            jnp.dot(h2, wf_ref[...], preferred_element_type=jnp.float32)
            + bf_ref[...]
        )

    return pl.pallas_call(
        body,
        grid=(NP // BR,),
        in_specs=[
            pl.BlockSpec((8, NC, BR, 4), lambda i: (0, 0, i, 0)),
            pl.BlockSpec((8, BR, 4), lambda i: (0, i, 0)),
            pl.BlockSpec((BR, 1), lambda i: (i, 0)),
            pl.BlockSpec((32, 32), lambda i: (0, 0)),
            pl.BlockSpec((1, 32), lambda i: (0, 0)),
            pl.BlockSpec((32, 1), lambda i: (0, 0)),
            pl.BlockSpec((1, 1), lambda i: (0, 0)),
        ],
        out_specs=pl.BlockSpec((BR, 1), lambda i: (i, 0)),
        out_shape=_f32((NP, 1)),
    )(u_part, h1_sl, dinv_col, W2, b2, Wfc, bfc)


_l1_kernel = _make_l1_kernel()
_l2_kernel = _make_l2_kernel(4, 8)


def kernel(x, c, ei, ew, W1, b1, W2, b2, Wfc, bfc):
    x = x.astype(jnp.float32)
    ew = ew.astype(jnp.float32)
    row = ei[0]
    col = ei[1]

    npad = EP - E
    pad_idx = jnp.arange(npad, dtype=jnp.int32) % N
    rowp = jnp.concatenate([row, pad_idx])
    colp = jnp.concatenate([col, pad_idx])
    ewp = jnp.concatenate([ew, jnp.zeros((npad,), jnp.float32)])

    x4 = jnp.zeros((NP, 4), jnp.float32).at[:N, :3].set(x)
    W1p = jnp.zeros((4, 32), jnp.float32).at[:3].set(W1)
    xf = jnp.zeros((NP,), jnp.float32)
    xf0 = xf.at[:N].set(x[:, 0])
    xf1 = xf.at[:N].set(x[:, 1])
    xf2 = xf.at[:N].set(x[:, 2])

    zd = jnp.zeros((WB,), jnp.float32)
    z4 = jnp.zeros((WB * 4,), jnp.float32)

    u1, wnorm, dinv = _l1_kernel(rowp, colp, ewp, xf0, xf1, xf2, zd)
    dinv_col = dinv.reshape(NP, 1)
    h1_sl = _mm1_tc(u1.reshape(NC, NP, 4), x4, dinv_col, W1p,
                    b1.reshape(1, 32))
    u2 = _l2_kernel(rowp, colp, wnorm, h1_sl.reshape(8 * NP * 4), z4)
    out = _mm2_tc(u2.reshape(8, NC, NP, 4), h1_sl, dinv_col, W2,
                  b2.reshape(1, 32), Wfc, bfc.reshape(1, 1))
    return out[:N]


# speedup vs baseline: 12.4886x; 2.7814x over previous
import functools

import jax
import jax.numpy as jnp
from jax import lax
from jax.experimental import pallas as pl
from jax.experimental.pallas import tpu as pltpu
from jax.experimental.pallas import tpu_sc as plsc

N = 50000
E = 1600000
NC, NS = 2, 16
NW = NC * NS
NP = 50176
NSLICE = NP // NS
CROWS = 8
CH = CROWS * 128
EW_PER = 50176
EP = NW * EW_PER
EROWS = EP // 128
NCHUNK = EW_PER // CH
DCHUNK = EP // NS // CH
WB = 784
BR = 1792

_mesh = functools.partial(
    plsc.VectorSubcoreMesh, core_axis_name="c", subcore_axis_name="s"
)


def _f32(shape):
    return jax.ShapeDtypeStruct(shape, jnp.float32)


def _newton_rsqrt(d):
    i = plsc.bitcast(d, jnp.int32)
    i = jnp.int32(0x5F3759DF) - lax.shift_right_logical(i, 1)
    y = plsc.bitcast(i, jnp.float32)
    for _ in range(3):
        y = y * (1.5 - 0.5 * d * y * y)
    return y


def _make_l1_kernel():
    scratch = [
        pltpu.VMEM((CROWS, 128), jnp.int32),
        pltpu.VMEM((CROWS, 128), jnp.int32),
        pltpu.VMEM((CROWS, 128), jnp.float32),
        pltpu.VMEM((CROWS, 128), jnp.float32),
        pltpu.VMEM((CROWS, 128), jnp.float32),
        pltpu.VMEM((NP,), jnp.float32),
        pltpu.VMEM((NP,), jnp.float32),
        pltpu.VMEM((WB,), jnp.float32),
        pltpu.VMEM_SHARED((NP,), jnp.float32),
        pltpu.VMEM_SHARED((NP,), jnp.float32),
        pltpu.VMEM_SHARED((NP,), jnp.float32),
    ]

    @functools.partial(
        pl.kernel,
        out_type=(_f32((3 * NC * NP,)), _f32((EROWS, 128)), _f32((NP,))),
        mesh=_mesh(),
        scratch_types=scratch,
        compiler_params=pltpu.CompilerParams(needs_layout_passes=False),
    )
    def l1_kernel(row_hbm, col_hbm, ew_hbm, xf0_hbm, xf1_hbm, xf2_hbm,
                  zd_hbm, u_hbm, wn_hbm, dinv_hbm,
                  colv, rowv, ewv, wnv, msgv, dinv_v, xcol, st0,
                  acc0, acc1, acc2):
        cc = lax.axis_index("c")
        sid = lax.axis_index("s")
        w = cc * NS + sid
        xf_refs = (xf0_hbm, xf1_hbm, xf2_hbm)
        accs = (acc0, acc1, acc2)

        pltpu.sync_copy(zd_hbm, st0)
        for j in range(NSLICE // WB):
            off = sid * NSLICE + j * WB
            for a in accs:
                pltpu.sync_copy(st0, a.at[pl.ds(off, WB)])
        plsc.subcore_barrier()

        def dchunk(i, carry):
            r0 = sid * (DCHUNK * CROWS) + i * CROWS
            pltpu.sync_copy(col_hbm.at[pl.ds(r0, CROWS), :], colv)
            pltpu.sync_copy(ew_hbm.at[pl.ds(r0, CROWS), :], ewv)
            for j in range(CROWS):
                pltpu.sync_copy(ewv.at[j], acc0.at[colv.at[j]], add=True)
            return carry

        lax.fori_loop(0, DCHUNK, dchunk, 0)
        plsc.subcore_barrier()

        for j in range(NSLICE // WB):
            off = sid * NSLICE + j * WB
            pltpu.sync_copy(acc0.at[pl.ds(off, WB)], st0)

            def dgroup(g, carry):
                d = st0[pl.ds(g * 16, 16)] + 1.0
                st0[pl.ds(g * 16, 16)] = _newton_rsqrt(d)
                return carry

            lax.fori_loop(0, WB // 16, dgroup, 0)
            pltpu.sync_copy(st0, dinv_hbm.at[pl.ds(off, WB)])
        pltpu.sync_copy(zd_hbm, st0)
        for j in range(NSLICE // WB):
            pltpu.sync_copy(st0, acc0.at[pl.ds(sid * NSLICE + j * WB, WB)])
        plsc.subcore_barrier()
        pltpu.sync_copy(dinv_hbm, dinv_v)

        for f in range(3):
            pltpu.sync_copy(xf_refs[f], xcol)

            def chunk(i, carry):
                r0 = w * (NCHUNK * CROWS) + i * CROWS
                pltpu.sync_copy(col_hbm.at[pl.ds(r0, CROWS), :], colv)
                pltpu.sync_copy(row_hbm.at[pl.ds(r0, CROWS), :], rowv)
                if f == 0:
                    pltpu.sync_copy(ew_hbm.at[pl.ds(r0, CROWS), :], ewv)
                else:
                    pltpu.sync_copy(wn_hbm.at[pl.ds(r0, CROWS), :], wnv)
                for j in range(CROWS):
                    for g in range(8):
                        rg = rowv[j, pl.ds(g * 16, 16)]
                        if f == 0:
                            cg = colv[j, pl.ds(g * 16, 16)]
                            eg = ewv[j, pl.ds(g * 16, 16)]
                            dr = plsc.load_gather(dinv_v, [rg])
                            dc = plsc.load_gather(dinv_v, [cg])
                            wn = dr * eg * dc
                            wnv[j, pl.ds(g * 16, 16)] = wn
                        else:
                            wn = wnv[j, pl.ds(g * 16, 16)]
                        xr = plsc.load_gather(xcol, [rg])
                        msgv[j, pl.ds(g * 16, 16)] = xr * wn
                if f == 0:
                    pltpu.sync_copy(wnv, wn_hbm.at[pl.ds(r0, CROWS), :])
                for j in range(CROWS):
                    pltpu.sync_copy(msgv.at[j], accs[f].at[colv.at[j]],
                                    add=True)
                return carry

            lax.fori_loop(0, NCHUNK, chunk, 0)
        plsc.subcore_barrier()

        for f in range(3):
            for j in range(NSLICE // WB):
                off = sid * NSLICE + j * WB
                pltpu.sync_copy(accs[f].at[pl.ds(off, WB)], st0)
                pltpu.sync_copy(
                    st0, u_hbm.at[pl.ds(f * NC * NP + cc * NP + off, WB)]
                )

    return l1_kernel


def _make_l2_kernel():
    scratch = [
        pltpu.VMEM((CROWS, 128), jnp.int32),
        pltpu.VMEM((CROWS, 128), jnp.int32),
        pltpu.VMEM((CROWS, 128), jnp.float32),
        pltpu.VMEM((4 * CROWS, 128), jnp.float32),
        pltpu.VMEM((4 * CROWS, 128), jnp.float32),
        pltpu.VMEM((WB,), jnp.float32),
        pltpu.VMEM_SHARED((NP,), jnp.float32),
        pltpu.VMEM_SHARED((NP,), jnp.float32),
        pltpu.VMEM_SHARED((NP,), jnp.float32),
        pltpu.VMEM_SHARED((NP,), jnp.float32),
        pltpu.VMEM_SHARED((NP,), jnp.float32),
        pltpu.VMEM_SHARED((NP,), jnp.float32),
        pltpu.VMEM_SHARED((NP,), jnp.float32),
        pltpu.VMEM_SHARED((NP,), jnp.float32),
        pltpu.SemaphoreType.DMA,
    ]

    @functools.partial(
        pl.kernel,
        out_type=_f32((32 * NC * NP,)),
        mesh=_mesh(),
        scratch_types=scratch,
        compiler_params=pltpu.CompilerParams(needs_layout_passes=False),
    )
    def l2_kernel(row_hbm, col_hbm, wn_hbm, src_hbm, zd_hbm, u_hbm,
                  colv, rowv, wnv, xg, msgv, st0,
                  s0, s1, s2, s3, a0, a1, a2, a3, sem):
        cc = lax.axis_index("c")
        sid = lax.axis_index("s")
        w = cc * NS + sid
        srcs = (s0, s1, s2, s3)
        accs = (a0, a1, a2, a3)

        def do_round(k, carry):
            pltpu.sync_copy(zd_hbm, st0)
            for j in range(NSLICE // WB):
                off = sid * NSLICE + j * WB
                for a in accs:
                    pltpu.sync_copy(st0, a.at[pl.ds(off, WB)])
            for f in range(4):
                for j in range(NSLICE // WB):
                    off = sid * NSLICE + j * WB
                    pltpu.sync_copy(
                        src_hbm.at[pl.ds((k * 4 + f) * NP + off, WB)], st0
                    )
                    pltpu.sync_copy(st0, srcs[f].at[pl.ds(off, WB)])
            plsc.subcore_barrier()

            def chunk(i, carry2):
                r0 = w * (NCHUNK * CROWS) + i * CROWS
                pltpu.sync_copy(col_hbm.at[pl.ds(r0, CROWS), :], colv)
                pltpu.sync_copy(row_hbm.at[pl.ds(r0, CROWS), :], rowv)
                pltpu.sync_copy(wn_hbm.at[pl.ds(r0, CROWS), :], wnv)
                copies = [
                    pltpu.async_copy(
                        srcs[f].at[rowv.at[j]], xg.at[f * CROWS + j], sem
                    )
                    for f in range(4)
                    for j in range(CROWS)
                ]
                for cp in copies:
                    cp.wait()
                for f in range(4):
                    for j in range(CROWS):
                        for g in range(8):
                            msgv[f * CROWS + j, pl.ds(g * 16, 16)] = (
                                xg[f * CROWS + j, pl.ds(g * 16, 16)]
                                * wnv[j, pl.ds(g * 16, 16)]
                            )
                for f in range(4):
                    for j in range(CROWS):
                        pltpu.sync_copy(
                            msgv.at[f * CROWS + j],
                            accs[f].at[colv.at[j]], add=True,
                        )
                return carry2

            lax.fori_loop(0, NCHUNK, chunk, 0)
            plsc.subcore_barrier()
            for f in range(4):
                for j in range(NSLICE // WB):
                    off = sid * NSLICE + j * WB
                    pltpu.sync_copy(accs[f].at[pl.ds(off, WB)], st0)
                    pltpu.sync_copy(
                        st0,
                        u_hbm.at[
                            pl.ds((k * 4 + f) * NC * NP + cc * NP + off, WB)
                        ],
                    )
            return carry

        lax.fori_loop(0, 8, do_round, 0)

    return l2_kernel


def _mm1_tc(u_part, xT, dinv_row, W1T, b1T):
    def body(u_ref, s_ref, d_ref, w_ref, b_ref, o_ref):
        d2 = d_ref[...] * d_ref[...]
        u = u_ref[:, 0, :] + u_ref[:, 1, :] + d2 * s_ref[...]
        o_ref[...] = jnp.maximum(
            jnp.dot(w_ref[...], u, preferred_element_type=jnp.float32)
            + b_ref[...],
            0.0,
        )

    return pl.pallas_call(
        body,
        grid=(NP // BR,),
        in_specs=[
            pl.BlockSpec((3, NC, BR), lambda i: (0, 0, i)),
            pl.BlockSpec((3, BR), lambda i: (0, i)),
            pl.BlockSpec((1, BR), lambda i: (0, i)),
            pl.BlockSpec((32, 3), lambda i: (0, 0)),
            pl.BlockSpec((32, 1), lambda i: (0, 0)),
        ],
        out_specs=pl.BlockSpec((32, BR), lambda i: (0, i)),
        out_shape=_f32((32, NP)),
    )(u_part, xT, dinv_row, W1T, b1T)


def _mm2_tc(u_part, h1T, dinv_row, W2T, b2T, WfcT, bfcT):
    def body(u_ref, s_ref, d_ref, w_ref, b_ref, wf_ref, bf_ref, o_ref):
        d2 = d_ref[...] * d_ref[...]
        u = u_ref[:, 0, :] + u_ref[:, 1, :] + d2 * s_ref[...]
        h2 = jnp.maximum(
            jnp.dot(w_ref[...], u, preferred_element_type=jnp.float32)
            + b_ref[...],
            0.0,
        )
        o_ref[...] = (
            jnp.dot(wf_ref[...], h2, preferred_element_type=jnp.float32)
            + bf_ref[...]
        )

    return pl.pallas_call(
        body,
        grid=(NP // BR,),
        in_specs=[
            pl.BlockSpec((32, NC, BR), lambda i: (0, 0, i)),
            pl.BlockSpec((32, BR), lambda i: (0, i)),
            pl.BlockSpec((1, BR), lambda i: (0, i)),
            pl.BlockSpec((32, 32), lambda i: (0, 0)),
            pl.BlockSpec((32, 1), lambda i: (0, 0)),
            pl.BlockSpec((1, 32), lambda i: (0, 0)),
            pl.BlockSpec((1, 1), lambda i: (0, 0)),
        ],
        out_specs=pl.BlockSpec((1, BR), lambda i: (0, i)),
        out_shape=_f32((1, NP)),
    )(u_part, h1T, dinv_row, W2T, b2T, WfcT, bfcT)


_l1_kernel = _make_l1_kernel()
_l2_kernel = _make_l2_kernel()


def kernel(x, c, ei, ew, W1, b1, W2, b2, Wfc, bfc):
    x = x.astype(jnp.float32)
    ew = ew.astype(jnp.float32)
    row = ei[0]
    col = ei[1]

    npad = EP - E
    pad_idx = jnp.arange(npad, dtype=jnp.int32) % N
    row2d = jnp.concatenate([row, pad_idx]).reshape(EROWS, 128)
    col2d = jnp.concatenate([col, pad_idx]).reshape(EROWS, 128)
    ew2d = jnp.concatenate(
        [ew, jnp.zeros((npad,), jnp.float32)]
    ).reshape(EROWS, 128)

    xT = jnp.zeros((3, NP), jnp.float32).at[:, :N].set(x.T)
    zd = jnp.zeros((WB,), jnp.float32)

    u1, wn2d, dinv = _l1_kernel(row2d, col2d, ew2d, xT[0], xT[1], xT[2], zd)
    dinv_row = dinv.reshape(1, NP)
    h1T = _mm1_tc(u1.reshape(3, NC, NP), xT, dinv_row, W1.T,
                  b1.reshape(32, 1))
    u2 = _l2_kernel(row2d, col2d, wn2d, h1T.reshape(32 * NP), zd)
    outT = _mm2_tc(u2.reshape(32, NC, NP), h1T, dinv_row, W2.T,
                   b2.reshape(32, 1), Wfc.T, bfc.reshape(1, 1))
    return outT.reshape(NP, 1)[:N]


# speedup vs baseline: 19.1788x; 1.5357x over previous
import functools

import jax
import jax.numpy as jnp
from jax import lax
from jax.experimental import pallas as pl
from jax.experimental.pallas import tpu as pltpu
from jax.experimental.pallas import tpu_sc as plsc

N = 50000
E = 1600000
NC, NS = 2, 16
NW = NC * NS
NP = 50176
NSLICE = NP // NS
CROWS = 8
CH = CROWS * 128
EW_PER = 51200
EP = NW * EW_PER
EROWS = EP // 128
NCHUNK = EW_PER // CH
DCHUNK = EP // NS // CH
WB = 784
BR = 1792

_mesh = functools.partial(
    plsc.VectorSubcoreMesh, core_axis_name="c", subcore_axis_name="s"
)


def _f32(shape):
    return jax.ShapeDtypeStruct(shape, jnp.float32)


def _newton_rsqrt(d):
    i = plsc.bitcast(d, jnp.int32)
    i = jnp.int32(0x5F3759DF) - lax.shift_right_logical(i, 1)
    y = plsc.bitcast(i, jnp.float32)
    for _ in range(3):
        y = y * (1.5 - 0.5 * d * y * y)
    return y


def _make_l1_kernel():
    scratch = [
        pltpu.VMEM((CROWS, 128), jnp.int32),
        pltpu.VMEM((CROWS, 128), jnp.int32),
        pltpu.VMEM((CROWS, 128), jnp.float32),
        pltpu.VMEM((CROWS, 128), jnp.float32),
        pltpu.VMEM((CROWS, 128), jnp.float32),
        pltpu.VMEM((CROWS, 128), jnp.int32),
        pltpu.VMEM((CROWS, 128), jnp.int32),
        pltpu.VMEM((CROWS, 128), jnp.float32),
        pltpu.VMEM((CROWS, 128), jnp.float32),
        pltpu.VMEM((CROWS, 128), jnp.float32),
        pltpu.VMEM((NP,), jnp.float32),
        pltpu.VMEM((NP,), jnp.float32),
        pltpu.VMEM((WB,), jnp.float32),
        pltpu.VMEM_SHARED((NP,), jnp.float32),
        pltpu.VMEM_SHARED((NP,), jnp.float32),
        pltpu.VMEM_SHARED((NP,), jnp.float32),
        pltpu.SemaphoreType.DMA,
    ]

    @functools.partial(
        pl.kernel,
        out_type=(_f32((3 * NC * NP,)), _f32((EROWS, 128)), _f32((NP,))),
        mesh=_mesh(),
        scratch_types=scratch,
        compiler_params=pltpu.CompilerParams(needs_layout_passes=False),
    )
    def l1_kernel(row_hbm, col_hbm, ew_hbm, xf0_hbm, xf1_hbm, xf2_hbm,
                  zd_hbm, u_hbm, wn_hbm, dinv_hbm,
                  colv, rowv, ewv, wnv, msgv, colv2, rowv2, ewv2, wnv2,
                  msgv2, dinv_v, xcol, st0, acc0, acc1, acc2, sem):
        cc = lax.axis_index("c")
        sid = lax.axis_index("s")
        w = cc * NS + sid
        xf_refs = (xf0_hbm, xf1_hbm, xf2_hbm)
        accs = (acc0, acc1, acc2)

        pltpu.sync_copy(zd_hbm, st0)
        for j in range(NSLICE // WB):
            off = sid * NSLICE + j * WB
            for a in accs:
                pltpu.sync_copy(st0, a.at[pl.ds(off, WB)])
        plsc.subcore_barrier()

        def dchunk(i, carry):
            r0 = sid * (DCHUNK * CROWS) + i * (2 * CROWS)
            pltpu.sync_copy(col_hbm.at[pl.ds(r0, CROWS), :], colv)
            pltpu.sync_copy(ew_hbm.at[pl.ds(r0, CROWS), :], ewv)
            descs = [
                pltpu.async_copy(ewv.at[j], acc0.at[colv.at[j]], sem,
                                 add=True)
                for j in range(CROWS)
            ]
            pltpu.sync_copy(col_hbm.at[pl.ds(r0 + CROWS, CROWS), :], colv2)
            pltpu.sync_copy(ew_hbm.at[pl.ds(r0 + CROWS, CROWS), :], ewv2)
            descs += [
                pltpu.async_copy(ewv2.at[j], acc0.at[colv2.at[j]], sem,
                                 add=True)
                for j in range(CROWS)
            ]
            for d in descs:
                d.wait()
            return carry

        lax.fori_loop(0, DCHUNK // 2, dchunk, 0)
        plsc.subcore_barrier()

        for j in range(NSLICE // WB):
            off = sid * NSLICE + j * WB
            pltpu.sync_copy(acc0.at[pl.ds(off, WB)], st0)

            def dgroup(g, carry):
                d = st0[pl.ds(g * 16, 16)] + 1.0
                st0[pl.ds(g * 16, 16)] = _newton_rsqrt(d)
                return carry

            lax.fori_loop(0, WB // 16, dgroup, 0)
            pltpu.sync_copy(st0, dinv_hbm.at[pl.ds(off, WB)])
        pltpu.sync_copy(zd_hbm, st0)
        for j in range(NSLICE // WB):
            pltpu.sync_copy(st0, acc0.at[pl.ds(sid * NSLICE + j * WB, WB)])
        plsc.subcore_barrier()
        pltpu.sync_copy(dinv_hbm, dinv_v)

        for f in range(3):
            pltpu.sync_copy(xf_refs[f], xcol)

            def sub(r0, cv, rv, ev, wv, mv):
                pltpu.sync_copy(col_hbm.at[pl.ds(r0, CROWS), :], cv)
                pltpu.sync_copy(row_hbm.at[pl.ds(r0, CROWS), :], rv)
                if f == 0:
                    pltpu.sync_copy(ew_hbm.at[pl.ds(r0, CROWS), :], ev)
                else:
                    pltpu.sync_copy(wn_hbm.at[pl.ds(r0, CROWS), :], wv)
                for j in range(CROWS):
                    for g in range(8):
                        rg = rv[j, pl.ds(g * 16, 16)]
                        if f == 0:
                            cg = cv[j, pl.ds(g * 16, 16)]
                            eg = ev[j, pl.ds(g * 16, 16)]
                            dr = plsc.load_gather(dinv_v, [rg])
                            dc = plsc.load_gather(dinv_v, [cg])
                            wn = dr * eg * dc
                            wv[j, pl.ds(g * 16, 16)] = wn
                        else:
                            wn = wv[j, pl.ds(g * 16, 16)]
                        xr = plsc.load_gather(xcol, [rg])
                        mv[j, pl.ds(g * 16, 16)] = xr * wn
                if f == 0:
                    pltpu.sync_copy(wv, wn_hbm.at[pl.ds(r0, CROWS), :])
                return [
                    pltpu.async_copy(mv.at[j], accs[f].at[cv.at[j]], sem,
                                     add=True)
                    for j in range(CROWS)
                ]

            def chunk(i, carry):
                r0 = w * (NCHUNK * CROWS) + i * (2 * CROWS)
                descs = sub(r0, colv, rowv, ewv, wnv, msgv)
                descs += sub(r0 + CROWS, colv2, rowv2, ewv2, wnv2, msgv2)
                for d in descs:
                    d.wait()
                return carry

            lax.fori_loop(0, NCHUNK // 2, chunk, 0)
        plsc.subcore_barrier()

        for f in range(3):
            for j in range(NSLICE // WB):
                off = sid * NSLICE + j * WB
                pltpu.sync_copy(accs[f].at[pl.ds(off, WB)], st0)
                pltpu.sync_copy(
                    st0, u_hbm.at[pl.ds(f * NC * NP + cc * NP + off, WB)]
                )

    return l1_kernel


def _make_l2_kernel():
    scratch = [
        pltpu.VMEM((CROWS, 128), jnp.int32),
        pltpu.VMEM((CROWS, 128), jnp.int32),
        pltpu.VMEM((CROWS, 128), jnp.float32),
        pltpu.VMEM((4 * CROWS, 128), jnp.float32),
        pltpu.VMEM((4 * CROWS, 128), jnp.float32),
        pltpu.VMEM((CROWS, 128), jnp.int32),
        pltpu.VMEM((CROWS, 128), jnp.int32),
        pltpu.VMEM((CROWS, 128), jnp.float32),
        pltpu.VMEM((4 * CROWS, 128), jnp.float32),
        pltpu.VMEM((4 * CROWS, 128), jnp.float32),
        pltpu.VMEM((WB,), jnp.float32),
        pltpu.VMEM_SHARED((NP,), jnp.float32),
        pltpu.VMEM_SHARED((NP,), jnp.float32),
        pltpu.VMEM_SHARED((NP,), jnp.float32),
        pltpu.VMEM_SHARED((NP,), jnp.float32),
        pltpu.VMEM_SHARED((NP,), jnp.float32),
        pltpu.VMEM_SHARED((NP,), jnp.float32),
        pltpu.VMEM_SHARED((NP,), jnp.float32),
        pltpu.VMEM_SHARED((NP,), jnp.float32),
        pltpu.SemaphoreType.DMA,
        pltpu.SemaphoreType.DMA,
        pltpu.SemaphoreType.DMA,
    ]

    @functools.partial(
        pl.kernel,
        out_type=_f32((32 * NC * NP,)),
        mesh=_mesh(),
        scratch_types=scratch,
        compiler_params=pltpu.CompilerParams(needs_layout_passes=False),
    )
    def l2_kernel(row_hbm, col_hbm, wn_hbm, src_hbm, zd_hbm, u_hbm,
                  colv, rowv, wnv, xg, msgv, colv2, rowv2, wnv2, xg2, msgv2,
                  st0, s0, s1, s2, s3, a0, a1, a2, a3, semA, semB, semS):
        cc = lax.axis_index("c")
        sid = lax.axis_index("s")
        w = cc * NS + sid
        srcs = (s0, s1, s2, s3)
        accs = (a0, a1, a2, a3)

        def do_round(k, carry):
            pltpu.sync_copy(zd_hbm, st0)
            for j in range(NSLICE // WB):
                off = sid * NSLICE + j * WB
                for a in accs:
                    pltpu.sync_copy(st0, a.at[pl.ds(off, WB)])
            for f in range(4):
                for j in range(NSLICE // WB):
                    off = sid * NSLICE + j * WB
                    pltpu.sync_copy(
                        src_hbm.at[pl.ds((k * 4 + f) * NP + off, WB)], st0
                    )
                    pltpu.sync_copy(st0, srcs[f].at[pl.ds(off, WB)])
            plsc.subcore_barrier()

            def load_and_gather(r0, cv, rv, wv, xgb, sem):
                pltpu.sync_copy(col_hbm.at[pl.ds(r0, CROWS), :], cv)
                pltpu.sync_copy(row_hbm.at[pl.ds(r0, CROWS), :], rv)
                pltpu.sync_copy(wn_hbm.at[pl.ds(r0, CROWS), :], wv)
                return [
                    pltpu.async_copy(
                        srcs[f].at[rv.at[j]], xgb.at[f * CROWS + j], sem
                    )
                    for f in range(4)
                    for j in range(CROWS)
                ]

            def scale_and_scatter(gcopies, cv, wv, xgb, mvb):
                for cp in gcopies:
                    cp.wait()
                for f in range(4):
                    for j in range(CROWS):
                        for g in range(8):
                            mvb[f * CROWS + j, pl.ds(g * 16, 16)] = (
                                xgb[f * CROWS + j, pl.ds(g * 16, 16)]
                                * wv[j, pl.ds(g * 16, 16)]
                            )
                return [
                    pltpu.async_copy(
                        mvb.at[f * CROWS + j], accs[f].at[cv.at[j]], semS,
                        add=True,
                    )
                    for f in range(4)
                    for j in range(CROWS)
                ]

            def chunk(i, carry2):
                r0 = w * (NCHUNK * CROWS) + i * (2 * CROWS)
                ga = load_and_gather(r0, colv, rowv, wnv, xg, semA)
                gb = load_and_gather(r0 + CROWS, colv2, rowv2, wnv2, xg2,
                                     semB)
                descs = scale_and_scatter(ga, colv, wnv, xg, msgv)
                descs += scale_and_scatter(gb, colv2, wnv2, xg2, msgv2)
                for d in descs:
                    d.wait()
                return carry2

            lax.fori_loop(0, NCHUNK // 2, chunk, 0)
            plsc.subcore_barrier()
            for f in range(4):
                for j in range(NSLICE // WB):
                    off = sid * NSLICE + j * WB
                    pltpu.sync_copy(accs[f].at[pl.ds(off, WB)], st0)
                    pltpu.sync_copy(
                        st0,
                        u_hbm.at[
                            pl.ds((k * 4 + f) * NC * NP + cc * NP + off, WB)
                        ],
                    )
            return carry

        lax.fori_loop(0, 8, do_round, 0)

    return l2_kernel


def _mm1_tc(u_part, xT, dinv_row, W1T, b1T):
    def body(u_ref, s_ref, d_ref, w_ref, b_ref, o_ref):
        d2 = d_ref[...] * d_ref[...]
        u = u_ref[:, 0, :] + u_ref[:, 1, :] + d2 * s_ref[...]
        o_ref[...] = jnp.maximum(
            jnp.dot(w_ref[...], u, preferred_element_type=jnp.float32)
            + b_ref[...],
            0.0,
        )

    return pl.pallas_call(
        body,
        grid=(NP // BR,),
        in_specs=[
            pl.BlockSpec((3, NC, BR), lambda i: (0, 0, i)),
            pl.BlockSpec((3, BR), lambda i: (0, i)),
            pl.BlockSpec((1, BR), lambda i: (0, i)),
            pl.BlockSpec((32, 3), lambda i: (0, 0)),
            pl.BlockSpec((32, 1), lambda i: (0, 0)),
        ],
        out_specs=pl.BlockSpec((32, BR), lambda i: (0, i)),
        out_shape=_f32((32, NP)),
    )(u_part, xT, dinv_row, W1T, b1T)


def _mm2_tc(u_part, h1T, dinv_row, W2T, b2T, WfcT, bfcT):
    def body(u_ref, s_ref, d_ref, w_ref, b_ref, wf_ref, bf_ref, o_ref):
        d2 = d_ref[...] * d_ref[...]
        u = u_ref[:, 0, :] + u_ref[:, 1, :] + d2 * s_ref[...]
        h2 = jnp.maximum(
            jnp.dot(w_ref[...], u, preferred_element_type=jnp.float32)
            + b_ref[...],
            0.0,
        )
        o_ref[...] = (
            jnp.dot(wf_ref[...], h2, preferred_element_type=jnp.float32)
            + bf_ref[...]
        )

    return pl.pallas_call(
        body,
        grid=(NP // BR,),
        in_specs=[
            pl.BlockSpec((32, NC, BR), lambda i: (0, 0, i)),
            pl.BlockSpec((32, BR), lambda i: (0, i)),
            pl.BlockSpec((1, BR), lambda i: (0, i)),
            pl.BlockSpec((32, 32), lambda i: (0, 0)),
            pl.BlockSpec((32, 1), lambda i: (0, 0)),
            pl.BlockSpec((1, 32), lambda i: (0, 0)),
            pl.BlockSpec((1, 1), lambda i: (0, 0)),
        ],
        out_specs=pl.BlockSpec((1, BR), lambda i: (0, i)),
        out_shape=_f32((1, NP)),
    )(u_part, h1T, dinv_row, W2T, b2T, WfcT, bfcT)


_l1_kernel = _make_l1_kernel()
_l2_kernel = _make_l2_kernel()


def kernel(x, c, ei, ew, W1, b1, W2, b2, Wfc, bfc):
    x = x.astype(jnp.float32)
    ew = ew.astype(jnp.float32)
    row = ei[0]
    col = ei[1]

    npad = EP - E
    pad_idx = jnp.arange(npad, dtype=jnp.int32) % N
    row2d = jnp.concatenate([row, pad_idx]).reshape(EROWS, 128)
    col2d = jnp.concatenate([col, pad_idx]).reshape(EROWS, 128)
    ew2d = jnp.concatenate(
        [ew, jnp.zeros((npad,), jnp.float32)]
    ).reshape(EROWS, 128)

    xT = jnp.zeros((3, NP), jnp.float32).at[:, :N].set(x.T)
    zd = jnp.zeros((WB,), jnp.float32)

    u1, wn2d, dinv = _l1_kernel(row2d, col2d, ew2d, xT[0], xT[1], xT[2], zd)
    dinv_row = dinv.reshape(1, NP)
    h1T = _mm1_tc(u1.reshape(3, NC, NP), xT, dinv_row, W1.T,
                  b1.reshape(32, 1))
    u2 = _l2_kernel(row2d, col2d, wn2d, h1T.reshape(32 * NP), zd)
    outT = _mm2_tc(u2.reshape(32, NC, NP), h1T, dinv_row, W2.T,
                   b2.reshape(32, 1), Wfc.T, bfc.reshape(1, 1))
    return outT.reshape(NP, 1)[:N]


# speedup vs baseline: 20.9237x; 1.0910x over previous
import functools

import jax
import jax.numpy as jnp
from jax import lax
from jax.experimental import pallas as pl
from jax.experimental.pallas import tpu as pltpu
from jax.experimental.pallas import tpu_sc as plsc

N = 50000
E = 1600000
NC, NS = 2, 16
NW = NC * NS
NP = 50176
NSLICE = NP // NS
CROWS = 8
CH = CROWS * 128
EW_PER = 51200
EP = NW * EW_PER
EROWS = EP // 128
NCHUNK = EW_PER // CH
DCHUNK = EP // NS // CH
WB = 784
BR = 1792

_mesh = functools.partial(
    plsc.VectorSubcoreMesh, core_axis_name="c", subcore_axis_name="s"
)


def _f32(shape):
    return jax.ShapeDtypeStruct(shape, jnp.float32)


def _newton_rsqrt(d):
    i = plsc.bitcast(d, jnp.int32)
    i = jnp.int32(0x5F3759DF) - lax.shift_right_logical(i, 1)
    y = plsc.bitcast(i, jnp.float32)
    for _ in range(3):
        y = y * (1.5 - 0.5 * d * y * y)
    return y


def _make_l1_kernel():
    scratch = [
        pltpu.VMEM((CROWS, 128), jnp.int32),
        pltpu.VMEM((CROWS, 128), jnp.int32),
        pltpu.VMEM((CROWS, 128), jnp.float32),
        pltpu.VMEM((CROWS, 128), jnp.float32),
        pltpu.VMEM((CROWS, 128), jnp.float32),
        pltpu.VMEM((CROWS, 128), jnp.int32),
        pltpu.VMEM((CROWS, 128), jnp.int32),
        pltpu.VMEM((CROWS, 128), jnp.float32),
        pltpu.VMEM((CROWS, 128), jnp.float32),
        pltpu.VMEM((CROWS, 128), jnp.float32),
        pltpu.VMEM((NP,), jnp.float32),
        pltpu.VMEM((NP,), jnp.float32),
        pltpu.VMEM((WB,), jnp.float32),
        pltpu.VMEM_SHARED((NP,), jnp.float32),
        pltpu.VMEM_SHARED((NP,), jnp.float32),
        pltpu.VMEM_SHARED((NP,), jnp.float32),
        pltpu.SemaphoreType.DMA,
    ]

    @functools.partial(
        pl.kernel,
        out_type=(_f32((3 * NC * NP,)), _f32((EROWS, 128)), _f32((NP,))),
        mesh=_mesh(),
        scratch_types=scratch,
        compiler_params=pltpu.CompilerParams(needs_layout_passes=False),
    )
    def l1_kernel(row_hbm, col_hbm, ew_hbm, xf0_hbm, xf1_hbm, xf2_hbm,
                  zd_hbm, u_hbm, wn_hbm, dinv_hbm,
                  colv, rowv, ewv, wnv, msgv, colv2, rowv2, ewv2, wnv2,
                  msgv2, dinv_v, xcol, st0, acc0, acc1, acc2, sem):
        cc = lax.axis_index("c")
        sid = lax.axis_index("s")
        w = cc * NS + sid
        xf_refs = (xf0_hbm, xf1_hbm, xf2_hbm)
        accs = (acc0, acc1, acc2)

        pltpu.sync_copy(zd_hbm, st0)
        for j in range(NSLICE // WB):
            off = sid * NSLICE + j * WB
            for a in accs:
                pltpu.sync_copy(st0, a.at[pl.ds(off, WB)])
        plsc.subcore_barrier()

        def dchunk(i, carry):
            r0 = sid * (DCHUNK * CROWS) + i * (2 * CROWS)
            pltpu.sync_copy(col_hbm.at[pl.ds(r0, CROWS), :], colv)
            pltpu.sync_copy(ew_hbm.at[pl.ds(r0, CROWS), :], ewv)
            descs = [
                pltpu.async_copy(ewv.at[j], acc0.at[colv.at[j]], sem,
                                 add=True)
                for j in range(CROWS)
            ]
            pltpu.sync_copy(col_hbm.at[pl.ds(r0 + CROWS, CROWS), :], colv2)
            pltpu.sync_copy(ew_hbm.at[pl.ds(r0 + CROWS, CROWS), :], ewv2)
            descs += [
                pltpu.async_copy(ewv2.at[j], acc0.at[colv2.at[j]], sem,
                                 add=True)
                for j in range(CROWS)
            ]
            for d in descs:
                d.wait()
            return carry

        lax.fori_loop(0, DCHUNK // 2, dchunk, 0)
        plsc.subcore_barrier()

        for j in range(NSLICE // WB):
            off = sid * NSLICE + j * WB
            pltpu.sync_copy(acc0.at[pl.ds(off, WB)], st0)

            def dgroup(g, carry):
                d = st0[pl.ds(g * 16, 16)] + 1.0
                st0[pl.ds(g * 16, 16)] = _newton_rsqrt(d)
                return carry

            lax.fori_loop(0, WB // 16, dgroup, 0)
            pltpu.sync_copy(st0, dinv_hbm.at[pl.ds(off, WB)])
        pltpu.sync_copy(zd_hbm, st0)
        for j in range(NSLICE // WB):
            pltpu.sync_copy(st0, acc0.at[pl.ds(sid * NSLICE + j * WB, WB)])
        plsc.subcore_barrier()
        pltpu.sync_copy(dinv_hbm, dinv_v)

        for p in range(2):
            if p == 0:
                pltpu.sync_copy(xf_refs[0], xcol)
            else:
                pltpu.sync_copy(xf_refs[1], xcol)
                pltpu.sync_copy(xf_refs[2], dinv_v)

            def sub(r0, cv, rv, ev, wv, mv):
                pltpu.sync_copy(col_hbm.at[pl.ds(r0, CROWS), :], cv)
                pltpu.sync_copy(row_hbm.at[pl.ds(r0, CROWS), :], rv)
                if p == 0:
                    pltpu.sync_copy(ew_hbm.at[pl.ds(r0, CROWS), :], ev)
                else:
                    pltpu.sync_copy(wn_hbm.at[pl.ds(r0, CROWS), :], wv)
                for j in range(CROWS):
                    for g in range(8):
                        rg = rv[j, pl.ds(g * 16, 16)]
                        if p == 0:
                            cg = cv[j, pl.ds(g * 16, 16)]
                            eg = ev[j, pl.ds(g * 16, 16)]
                            dr = plsc.load_gather(dinv_v, [rg])
                            dc = plsc.load_gather(dinv_v, [cg])
                            wn = dr * eg * dc
                            wv[j, pl.ds(g * 16, 16)] = wn
                            xr = plsc.load_gather(xcol, [rg])
                            mv[j, pl.ds(g * 16, 16)] = xr * wn
                        else:
                            wn = wv[j, pl.ds(g * 16, 16)]
                            xr = plsc.load_gather(xcol, [rg])
                            mv[j, pl.ds(g * 16, 16)] = xr * wn
                            x2 = plsc.load_gather(dinv_v, [rg])
                            ev[j, pl.ds(g * 16, 16)] = x2 * wn
                if p == 0:
                    pltpu.sync_copy(wv, wn_hbm.at[pl.ds(r0, CROWS), :])
                descs = [
                    pltpu.async_copy(
                        mv.at[j], accs[0 if p == 0 else 1].at[cv.at[j]],
                        sem, add=True)
                    for j in range(CROWS)
                ]
                if p == 1:
                    descs += [
                        pltpu.async_copy(ev.at[j], accs[2].at[cv.at[j]],
                                         sem, add=True)
                        for j in range(CROWS)
                    ]
                return descs

            def chunk(i, carry):
                r0 = w * (NCHUNK * CROWS) + i * (2 * CROWS)
                descs = sub(r0, colv, rowv, ewv, wnv, msgv)
                descs += sub(r0 + CROWS, colv2, rowv2, ewv2, wnv2, msgv2)
                for d in descs:
                    d.wait()
                return carry

            lax.fori_loop(0, NCHUNK // 2, chunk, 0)
        plsc.subcore_barrier()

        for f in range(3):
            for j in range(NSLICE // WB):
                off = sid * NSLICE + j * WB
                pltpu.sync_copy(accs[f].at[pl.ds(off, WB)], st0)
                pltpu.sync_copy(
                    st0, u_hbm.at[pl.ds(f * NC * NP + cc * NP + off, WB)]
                )

    return l1_kernel


def _make_l2_kernel():
    scratch = [
        pltpu.VMEM((CROWS, 128), jnp.int32),
        pltpu.VMEM((CROWS, 128), jnp.int32),
        pltpu.VMEM((CROWS, 128), jnp.float32),
        pltpu.VMEM((4 * CROWS, 128), jnp.float32),
        pltpu.VMEM((4 * CROWS, 128), jnp.float32),
        pltpu.VMEM((CROWS, 128), jnp.int32),
        pltpu.VMEM((CROWS, 128), jnp.int32),
        pltpu.VMEM((CROWS, 128), jnp.float32),
        pltpu.VMEM((4 * CROWS, 128), jnp.float32),
        pltpu.VMEM((4 * CROWS, 128), jnp.float32),
        pltpu.VMEM((WB,), jnp.float32),
        pltpu.VMEM_SHARED((NP,), jnp.float32),
        pltpu.VMEM_SHARED((NP,), jnp.float32),
        pltpu.VMEM_SHARED((NP,), jnp.float32),
        pltpu.VMEM_SHARED((NP,), jnp.float32),
        pltpu.VMEM_SHARED((NP,), jnp.float32),
        pltpu.VMEM_SHARED((NP,), jnp.float32),
        pltpu.VMEM_SHARED((NP,), jnp.float32),
        pltpu.VMEM_SHARED((NP,), jnp.float32),
        pltpu.SemaphoreType.DMA,
        pltpu.SemaphoreType.DMA,
        pltpu.SemaphoreType.DMA,
        pltpu.SemaphoreType.DMA,
    ]

    @functools.partial(
        pl.kernel,
        out_type=_f32((32 * NC * NP,)),
        mesh=_mesh(),
        scratch_types=scratch,
        compiler_params=pltpu.CompilerParams(needs_layout_passes=False),
    )
    def l2_kernel(row_hbm, col_hbm, wn_hbm, src_hbm, zd_hbm, zdummy_hbm,
                  u_hbm,
                  colv, rowv, wnv, xg, msgv, colv2, rowv2, wnv2, xg2, msgv2,
                  st0, s0, s1, s2, s3, a0, a1, a2, a3,
                  semA, semB, semSA, semSB):
        cc = lax.axis_index("c")
        sid = lax.axis_index("s")
        w = cc * NS + sid
        srcs = (s0, s1, s2, s3)
        accs = (a0, a1, a2, a3)

        def do_round(k, carry):
            pltpu.sync_copy(zd_hbm, st0)
            for j in range(NSLICE // WB):
                off = sid * NSLICE + j * WB
                for a in accs:
                    pltpu.sync_copy(st0, a.at[pl.ds(off, WB)])
            for f in range(4):
                for j in range(NSLICE // WB):
                    off = sid * NSLICE + j * WB
                    pltpu.sync_copy(
                        src_hbm.at[pl.ds((k * 4 + f) * NP + off, WB)], st0
                    )
                    pltpu.sync_copy(st0, srcs[f].at[pl.ds(off, WB)])
            plsc.subcore_barrier()

            def load_and_gather(r0, cv, rv, wv, xgb, sem):
                pltpu.sync_copy(col_hbm.at[pl.ds(r0, CROWS), :], cv)
                pltpu.sync_copy(row_hbm.at[pl.ds(r0, CROWS), :], rv)
                pltpu.sync_copy(wn_hbm.at[pl.ds(r0, CROWS), :], wv)
                return [
                    pltpu.async_copy(
                        srcs[f].at[rv.at[j]], xgb.at[f * CROWS + j], sem
                    )
                    for f in range(4)
                    for j in range(CROWS)
                ]

            def scale_and_scatter(gcopies, cv, wv, xgb, mvb, semS):
                for cp in gcopies:
                    cp.wait()
                for f in range(4):
                    for j in range(CROWS):
                        for g in range(8):
                            mvb[f * CROWS + j, pl.ds(g * 16, 16)] = (
                                xgb[f * CROWS + j, pl.ds(g * 16, 16)]
                                * wv[j, pl.ds(g * 16, 16)]
                            )
                for f in range(4):
                    for j in range(CROWS):
                        pltpu.async_copy(
                            mvb.at[f * CROWS + j], accs[f].at[cv.at[j]],
                            semS, add=True,
                        )

            def drain(mvb, semS):
                pltpu.make_async_copy(zdummy_hbm, mvb, semS).wait()

            def chunk(i, carry2):
                r0 = w * (NCHUNK * CROWS) + i * (2 * CROWS)

                @pl.when(i > 0)
                def _():
                    drain(msgv, semSA)

                ga = load_and_gather(r0, colv, rowv, wnv, xg, semA)

                @pl.when(i > 0)
                def _():
                    drain(msgv2, semSB)

                gb = load_and_gather(r0 + CROWS, colv2, rowv2, wnv2, xg2,
                                     semB)
                scale_and_scatter(ga, colv, wnv, xg, msgv, semSA)
                scale_and_scatter(gb, colv2, wnv2, xg2, msgv2, semSB)
                return carry2

            lax.fori_loop(0, NCHUNK // 2, chunk, 0)
            drain(msgv, semSA)
            drain(msgv2, semSB)
            plsc.subcore_barrier()
            for f in range(4):
                for j in range(NSLICE // WB):
                    off = sid * NSLICE + j * WB
                    pltpu.sync_copy(accs[f].at[pl.ds(off, WB)], st0)
                    pltpu.sync_copy(
                        st0,
                        u_hbm.at[
                            pl.ds((k * 4 + f) * NC * NP + cc * NP + off, WB)
                        ],
                    )
            return carry

        lax.fori_loop(0, 8, do_round, 0)

    return l2_kernel


def _mm1_tc(u_part, xT, dinv_row, W1T, b1T):
    def body(u_ref, s_ref, d_ref, w_ref, b_ref, o_ref):
        d2 = d_ref[...] * d_ref[...]
        u = u_ref[:, 0, :] + u_ref[:, 1, :] + d2 * s_ref[...]
        o_ref[...] = jnp.maximum(
            jnp.dot(w_ref[...], u, preferred_element_type=jnp.float32)
            + b_ref[...],
            0.0,
        )

    return pl.pallas_call(
        body,
        grid=(NP // BR,),
        in_specs=[
            pl.BlockSpec((3, NC, BR), lambda i: (0, 0, i)),
            pl.BlockSpec((3, BR), lambda i: (0, i)),
            pl.BlockSpec((1, BR), lambda i: (0, i)),
            pl.BlockSpec((32, 3), lambda i: (0, 0)),
            pl.BlockSpec((32, 1), lambda i: (0, 0)),
        ],
        out_specs=pl.BlockSpec((32, BR), lambda i: (0, i)),
        out_shape=_f32((32, NP)),
    )(u_part, xT, dinv_row, W1T, b1T)


def _mm2_tc(u_part, h1T, dinv_row, W2T, b2T, WfcT, bfcT):
    def body(u_ref, s_ref, d_ref, w_ref, b_ref, wf_ref, bf_ref, o_ref):
        d2 = d_ref[...] * d_ref[...]
        u = u_ref[:, 0, :] + u_ref[:, 1, :] + d2 * s_ref[...]
        h2 = jnp.maximum(
            jnp.dot(w_ref[...], u, preferred_element_type=jnp.float32)
            + b_ref[...],
            0.0,
        )
        o_ref[...] = (
            jnp.dot(wf_ref[...], h2, preferred_element_type=jnp.float32)
            + bf_ref[...]
        )

    return pl.pallas_call(
        body,
        grid=(NP // BR,),
        in_specs=[
            pl.BlockSpec((32, NC, BR), lambda i: (0, 0, i)),
            pl.BlockSpec((32, BR), lambda i: (0, i)),
            pl.BlockSpec((1, BR), lambda i: (0, i)),
            pl.BlockSpec((32, 32), lambda i: (0, 0)),
            pl.BlockSpec((32, 1), lambda i: (0, 0)),
            pl.BlockSpec((1, 32), lambda i: (0, 0)),
            pl.BlockSpec((1, 1), lambda i: (0, 0)),
        ],
        out_specs=pl.BlockSpec((1, BR), lambda i: (0, i)),
        out_shape=_f32((1, NP)),
    )(u_part, h1T, dinv_row, W2T, b2T, WfcT, bfcT)


_l1_kernel = _make_l1_kernel()
_l2_kernel = _make_l2_kernel()


def kernel(x, c, ei, ew, W1, b1, W2, b2, Wfc, bfc):
    x = x.astype(jnp.float32)
    ew = ew.astype(jnp.float32)
    row = ei[0]
    col = ei[1]

    npad = EP - E
    pad_idx = jnp.arange(npad, dtype=jnp.int32) % N
    row2d = jnp.concatenate([row, pad_idx]).reshape(EROWS, 128)
    col2d = jnp.concatenate([col, pad_idx]).reshape(EROWS, 128)
    ew2d = jnp.concatenate(
        [ew, jnp.zeros((npad,), jnp.float32)]
    ).reshape(EROWS, 128)

    xT = jnp.zeros((3, NP), jnp.float32).at[:, :N].set(x.T)
    zd = jnp.zeros((WB,), jnp.float32)

    u1, wn2d, dinv = _l1_kernel(row2d, col2d, ew2d, xT[0], xT[1], xT[2], zd)
    dinv_row = dinv.reshape(1, NP)
    h1T = _mm1_tc(u1.reshape(3, NC, NP), xT, dinv_row, W1.T,
                  b1.reshape(32, 1))
    zdummy = jnp.zeros((4 * CROWS, 128), jnp.float32)
    u2 = _l2_kernel(row2d, col2d, wn2d, h1T.reshape(32 * NP), zd, zdummy)
    outT = _mm2_tc(u2.reshape(32, NC, NP), h1T, dinv_row, W2.T,
                   b2.reshape(32, 1), Wfc.T, bfc.reshape(1, 1))
    return outT.reshape(NP, 1)[:N]
